# baseline (speedup 1.0000x reference)
import functools

import jax
import jax.numpy as jnp
from jax import lax
from jax.experimental import pallas as pl
from jax.experimental.pallas import tpu as pltpu

_B, _C, _H, _W = 4, 384, 224, 224
_HW = _H * _W
_K = int(0.7 * _C)

_POOL_CHUNK = 3584
_APPLY_CHUNK = 1024


def _pool_mlp_body(nsteps, feat_ref, w1_ref, b1_ref, w2_ref, b2_ref,
                   soft_ref, mask_ref, acc_ref):
    j = pl.program_id(0)
    part = jnp.sum(feat_ref[...], axis=2)

    @pl.when(j == 0)
    def _():
        acc_ref[...] = part

    @pl.when(j != 0)
    def _():
        acc_ref[...] = acc_ref[...] + part

    @pl.when(j == nsteps - 1)
    def _():
        pooled = acc_ref[...] * (1.0 / _HW)
        h = lax.dot_general(pooled, w1_ref[...],
                            (((1,), (1,)), ((), ())),
                            preferred_element_type=jnp.float32)
        h = jnp.maximum(h + b1_ref[...][None, :], 0.0)
        z = lax.dot_general(h, w2_ref[...],
                            (((1,), (1,)), ((), ())),
                            preferred_element_type=jnp.float32)
        soft = jax.nn.sigmoid(z + b2_ref[...][None, :])
        soft_ref[...] = soft
        vi = soft[:, :, None]
        vj = soft[:, None, :]
        ii = lax.broadcasted_iota(jnp.int32, (_B, _C, _C), 1)
        jj = lax.broadcasted_iota(jnp.int32, (_B, _C, _C), 2)
        beats = (vj > vi) | ((vj == vi) & (jj < ii))
        rank = jnp.sum(beats.astype(jnp.int32), axis=2)
        mask_ref[...] = (rank < _K).astype(jnp.float32)


def _apply_body(feat_ref, mask_ref, causal_ref, noncausal_ref):
    m = mask_ref[...][:, :, None]
    f = feat_ref[...]
    c = f * m
    causal_ref[...] = c
    noncausal_ref[...] = f - c


@jax.jit
def kernel(feat, w1, b1, w2, b2):
    f3 = feat.reshape(_B, _C, _HW)
    nsteps = _HW // _POOL_CHUNK
    soft_mask, mask = pl.pallas_call(
        functools.partial(_pool_mlp_body, nsteps),
        grid=(nsteps,),
        in_specs=[
            pl.BlockSpec((_B, _C, _POOL_CHUNK), lambda j: (0, 0, j)),
            pl.BlockSpec((_C, _C), lambda j: (0, 0)),
            pl.BlockSpec((_C,), lambda j: (0,)),
            pl.BlockSpec((_C, _C), lambda j: (0, 0)),
            pl.BlockSpec((_C,), lambda j: (0,)),
        ],
        out_specs=[
            pl.BlockSpec((_B, _C), lambda j: (0, 0)),
            pl.BlockSpec((_B, _C), lambda j: (0, 0)),
        ],
        out_shape=[
            jax.ShapeDtypeStruct((_B, _C), jnp.float32),
            jax.ShapeDtypeStruct((_B, _C), jnp.float32),
        ],
        scratch_shapes=[pltpu.VMEM((_B, _C), jnp.float32)],
    )(f3, w1, b1, w2, b2)

    napply = _HW // _APPLY_CHUNK
    causal, noncausal = pl.pallas_call(
        _apply_body,
        grid=(napply,),
        in_specs=[
            pl.BlockSpec((_B, _C, _APPLY_CHUNK), lambda j: (0, 0, j)),
            pl.BlockSpec((_B, _C), lambda j: (0, 0)),
        ],
        out_specs=[
            pl.BlockSpec((_B, _C, _APPLY_CHUNK), lambda j: (0, 0, j)),
            pl.BlockSpec((_B, _C, _APPLY_CHUNK), lambda j: (0, 0, j)),
        ],
        out_shape=[
            jax.ShapeDtypeStruct((_B, _C, _HW), jnp.float32),
            jax.ShapeDtypeStruct((_B, _C, _HW), jnp.float32),
        ],
    )(f3, mask)

    causal = causal.reshape(_B, _C, _H, _W)
    noncausal = noncausal.reshape(_B, _C, _H, _W)
    mask4 = mask.reshape(_B, _C, 1, 1)
    return (causal, noncausal, mask4, soft_mask)
